# baseline (speedup 1.0000x reference)
import functools

import jax
import jax.numpy as jnp
from jax import lax
from jax.experimental import pallas as pl
from jax.experimental.pallas import tpu as pltpu
from jax.experimental.pallas import tpu_sc as plsc

_NC = 2
_NS = 16
_CHUNK = 448
_G = 64
_NP = 51200
_PR = _NP // 4


def _mesh():
    return plsc.VectorSubcoreMesh(
        core_axis_name="c", subcore_axis_name="s", num_cores=_NC,
        num_subcores=_NS)


def _fill_zeros_1d(ref, n):
    def body(i, _):
        ref[pl.ds(i * 16, 16)] = jnp.zeros((16,), jnp.float32)
        return 0
    lax.fori_loop(0, n // 16, body, 0)


def _fill_zeros_2d(ref, rows, w):
    def body(i, _):
        for k in range(w // 16):
            ref[i, pl.ds(k * 16, 16)] = jnp.zeros((16,), jnp.float32)
        return 0
    lax.fori_loop(0, rows, body, 0)


def _sc_degree(dst_pad, n_pad):
    e_pad = dst_pad.shape[0]
    per_tile = e_pad // (_NC * _NS)
    n_chunks = per_tile // _CHUNK
    stripe = n_pad // _NS
    zrows = 800
    assert stripe % zrows == 0 and per_tile % _CHUNK == 0

    @functools.partial(
        pl.kernel,
        out_type=jax.ShapeDtypeStruct((_NC, n_pad), jnp.float32),
        mesh=_mesh(),
        scratch_types=[
            pltpu.VMEM((2, _CHUNK), jnp.int32),
            pltpu.VMEM((_CHUNK,), jnp.float32),
            pltpu.VMEM((zrows,), jnp.float32),
            pltpu.VMEM_SHARED((n_pad,), jnp.float32),
            pltpu.SemaphoreType.DMA,
            pltpu.SemaphoreType.DMA,
        ],
        compiler_params=pltpu.CompilerParams(use_tc_tiling_on_sc=False),
    )
    def k(dst_hbm, out_hbm, idx_v, ones_v, zbuf_v, acc_sh, si0, si1):
        c = lax.axis_index("c")
        s = lax.axis_index("s")
        sem_i = [si0, si1]
        for i in range(_CHUNK // 16):
            ones_v[pl.ds(i * 16, 16)] = jnp.ones((16,), jnp.float32)
        _fill_zeros_1d(zbuf_v, zrows)
        row0 = s * stripe

        def zloop(j, _):
            pltpu.sync_copy(zbuf_v, acc_sh.at[pl.ds(row0 + j * zrows, zrows)])
            return 0
        lax.fori_loop(0, stripe // zrows, zloop, 0)
        plsc.subcore_barrier()

        base = (c * _NS + s) * per_tile
        npairs = n_chunks // 2
        assert n_chunks % 2 == 0

        def issue_idx(cc, b):
            pltpu.async_copy(dst_hbm.at[pl.ds(base + cc * _CHUNK, _CHUNK)],
                             idx_v.at[b], sem_i[b])

        def wait_idx(cc, b):
            pltpu.make_async_copy(
                dst_hbm.at[pl.ds(base + cc * _CHUNK, _CHUNK)],
                idx_v.at[b], sem_i[b]).wait()

        issue_idx(0, 0)
        issue_idx(1, 1)

        def body(g, _):
            for half in range(2):
                cc = 2 * g + half
                b = half
                wait_idx(cc, b)
                pltpu.sync_copy(ones_v, acc_sh.at[idx_v.at[b]], add=True)

                @pl.when(g < npairs - 1)
                def _():
                    issue_idx(cc + 2, b)
            return 0
        lax.fori_loop(0, npairs, body, 0)
        plsc.subcore_barrier()
        pltpu.sync_copy(acc_sh.at[pl.ds(row0, stripe)],
                        out_hbm.at[c, pl.ds(row0, stripe)])

    return k(dst_pad)


def _sc_aggregate(src_pad, dst_pad, table, chunk, fsplit):
    e_pad = src_pad.shape[0]
    if fsplit:
        _, n_pad, w = table.shape
        per_tile = e_pad // _NS
    else:
        n_pad, w = table.shape
        per_tile = e_pad // (_NC * _NS)
    n_chunks = per_tile // chunk
    npairs = n_chunks // 2
    stripe = n_pad // _NS
    zrows = 200
    assert stripe % zrows == 0 and per_tile % chunk == 0 and n_chunks % 2 == 0

    @functools.partial(
        pl.kernel,
        out_type=jax.ShapeDtypeStruct((_NC, n_pad, w), jnp.float32),
        mesh=_mesh(),
        scratch_types=[
            pltpu.VMEM((2, chunk), jnp.int32),
            pltpu.VMEM((2, chunk), jnp.int32),
            pltpu.VMEM((2, chunk, w), jnp.float32),
            pltpu.VMEM((zrows, w), jnp.float32),
            pltpu.VMEM_SHARED((n_pad, w), jnp.float32),
            pltpu.SemaphoreType.DMA,
            pltpu.SemaphoreType.DMA,
            pltpu.SemaphoreType.DMA,
            pltpu.SemaphoreType.DMA,
        ],
        compiler_params=pltpu.CompilerParams(use_tc_tiling_on_sc=False),
    )
    def k(src_hbm, dst_hbm, tab_hbm, out_hbm,
          sidx_v, didx_v, rows_v, zbuf_v, acc_sh, sg0, sg1, si0, si1):
        c = lax.axis_index("c")
        s = lax.axis_index("s")
        sem_g = [sg0, sg1]
        sem_i = [si0, si1]
        tab = tab_hbm.at[c] if fsplit else tab_hbm
        _fill_zeros_2d(zbuf_v, zrows, w)
        row0 = s * stripe

        def zloop(j, _):
            pltpu.sync_copy(zbuf_v, acc_sh.at[pl.ds(row0 + j * zrows, zrows)])
            return 0
        lax.fori_loop(0, stripe // zrows, zloop, 0)
        plsc.subcore_barrier()

        base = (s if fsplit else c * _NS + s) * per_tile

        def issue_idx(cc, b):
            off = base + cc * chunk
            pltpu.async_copy(src_hbm.at[pl.ds(off, chunk)], sidx_v.at[b],
                             sem_i[b])
            pltpu.async_copy(dst_hbm.at[pl.ds(off, chunk)], didx_v.at[b],
                             sem_i[b])

        def wait_idx(cc, b):
            off = base + cc * chunk
            pltpu.make_async_copy(src_hbm.at[pl.ds(off, chunk)],
                                  sidx_v.at[b], sem_i[b]).wait()
            pltpu.make_async_copy(dst_hbm.at[pl.ds(off, chunk)],
                                  didx_v.at[b], sem_i[b]).wait()

        def issue_gather(b):
            pltpu.async_copy(tab.at[sidx_v.at[b]], rows_v.at[b], sem_g[b])

        def wait_gather(b):
            pltpu.make_async_copy(tab.at[sidx_v.at[b]], rows_v.at[b],
                                  sem_g[b]).wait()

        issue_idx(0, 0)
        issue_idx(1, 1)
        wait_idx(0, 0)
        issue_gather(0)

        def body(g, _):
            for half in range(2):
                cc = 2 * g + half
                b = half
                nb = 1 - half
                wait_gather(b)
                if half == 0:
                    wait_idx(cc + 1, nb)
                    issue_gather(nb)
                else:
                    @pl.when(g < npairs - 1)
                    def _():
                        wait_idx(cc + 1, nb)
                        issue_gather(nb)
                pltpu.sync_copy(rows_v.at[b], acc_sh.at[didx_v.at[b]],
                                add=True)

                @pl.when(g < npairs - 1)
                def _():
                    issue_idx(cc + 2, b)
            return 0
        lax.fori_loop(0, npairs, body, 0)
        plsc.subcore_barrier()
        pltpu.sync_copy(acc_sh.at[pl.ds(row0, stripe)],
                        out_hbm.at[c, pl.ds(row0, stripe)])

    return k(src_pad, dst_pad, table)


def _tc_dinv(degp_pk):
    m = degp_pk.shape[0] // 2

    def body(d0_ref, d1_ref, o_ref):
        o_ref[...] = lax.rsqrt(d0_ref[...] + d1_ref[...] + 1.0)

    return pl.pallas_call(
        body,
        grid=(1,),
        in_specs=[
            pl.BlockSpec((m, 128), lambda i: (0, 0)),
            pl.BlockSpec((m, 128), lambda i: (1, 0)),
        ],
        out_specs=pl.BlockSpec((m, 128), lambda i: (0, 0)),
        out_shape=jax.ShapeDtypeStruct((m, 128), jnp.float32),
    )(degp_pk, degp_pk)


def _tc_stage_a(x4n, dinvrep, W1p):
    def body(x_ref, dv_ref, w_ref, o_ref):
        hs = [jnp.dot(x_ref[j], w_ref[...],
                      preferred_element_type=jnp.float32) for j in range(4)]
        o_ref[...] = jnp.concatenate(hs, axis=1) * dv_ref[...]

    return pl.pallas_call(
        body,
        grid=(_PR // 128,),
        in_specs=[
            pl.BlockSpec((4, 128, 8), lambda i: (0, i, 0)),
            pl.BlockSpec((128, 128), lambda i: (i, 0)),
            pl.BlockSpec((8, 32), lambda i: (0, 0)),
        ],
        out_specs=pl.BlockSpec((128, 128), lambda i: (i, 0)),
        out_shape=jax.ShapeDtypeStruct((_PR, 128), jnp.float32),
    )(x4n, dinvrep, W1p)


def _tc_stage_b(agg1_pk, u1_pk, dinvrep, b1, W2):
    def body(a_ref, u_ref, dv_ref, b_ref, w_ref, o_ref):
        dv = dv_ref[...]
        b1p = jnp.concatenate([b_ref[...]] * 4, axis=1)
        h1p = jnp.maximum((a_ref[0] + a_ref[1] + u_ref[...]) * dv + b1p, 0.0)
        h2m = [jnp.dot(h1p[:, 32 * j:32 * j + 32], w_ref[...],
                       preferred_element_type=jnp.float32) for j in range(4)]
        p0 = jnp.concatenate([h[:, :32] for h in h2m], axis=1) * dv
        p1 = jnp.concatenate([h[:, 32:] for h in h2m], axis=1) * dv
        o_ref[0] = p0
        o_ref[1] = p1

    return pl.pallas_call(
        body,
        grid=(_PR // 128,),
        in_specs=[
            pl.BlockSpec((2, 128, 128), lambda i: (0, i, 0)),
            pl.BlockSpec((128, 128), lambda i: (i, 0)),
            pl.BlockSpec((128, 128), lambda i: (i, 0)),
            pl.BlockSpec((1, 32), lambda i: (0, 0)),
            pl.BlockSpec((32, 64), lambda i: (0, 0)),
        ],
        out_specs=pl.BlockSpec((2, 128, 128), lambda i: (0, i, 0)),
        out_shape=jax.ShapeDtypeStruct((2, _PR, 128), jnp.float32),
    )(agg1_pk, u1_pk, dinvrep, b1.reshape(1, -1), W2)


def _tc_stage_c(agg2_pk, u2_pk, dinvrep, batchrep, b2):
    def body(a_ref, u_ref, dv_ref, br_ref, b_ref, o_ref):
        dv = dv_ref[...]
        b2p0 = jnp.concatenate([b_ref[:, 0:32]] * 4, axis=1)
        b2p1 = jnp.concatenate([b_ref[:, 32:64]] * 4, axis=1)
        h0 = jnp.maximum((a_ref[0] + u_ref[0]) * dv + b2p0, 0.0)
        h1 = jnp.maximum((a_ref[1] + u_ref[1]) * dv + b2p1, 0.0)
        gids = lax.broadcasted_iota(jnp.int32, (128, _G), 1)
        ones = jnp.ones((128, 8), jnp.float32)
        contrib = jnp.zeros((_G, 72), jnp.float32)
        for j in range(4):
            hj = jnp.concatenate(
                [h0[:, 32 * j:32 * j + 32], h1[:, 32 * j:32 * j + 32], ones],
                axis=1)
            bcol = br_ref[:, 32 * j:32 * j + 1]
            ohj = (bcol == gids).astype(jnp.float32)
            contrib = contrib + lax.dot_general(
                ohj, hj, (((0,), (0,)), ((), ())),
                preferred_element_type=jnp.float32,
                precision=lax.Precision.HIGHEST)

        @pl.when(pl.program_id(0) == 0)
        def _():
            o_ref[...] = jnp.zeros((_G, 72), jnp.float32)
        o_ref[...] += contrib

    return pl.pallas_call(
        body,
        grid=(_PR // 128,),
        in_specs=[
            pl.BlockSpec((2, 128, 128), lambda i: (0, i, 0)),
            pl.BlockSpec((2, 128, 128), lambda i: (0, i, 0)),
            pl.BlockSpec((128, 128), lambda i: (i, 0)),
            pl.BlockSpec((128, 128), lambda i: (i, 0)),
            pl.BlockSpec((1, 64), lambda i: (0, 0)),
        ],
        out_specs=pl.BlockSpec((_G, 72), lambda i: (0, 0)),
        out_shape=jax.ShapeDtypeStruct((_G, 72), jnp.float32),
    )(agg2_pk, u2_pk, dinvrep, batchrep, b2.reshape(1, -1))


def _tc_head(sums_cnt, Wl1, bl1, Wl2, bl2):
    def body(sc_ref, w1_ref, b1_ref, w2_ref, b2_ref, o_ref):
        sums = sc_ref[:, 0:64]
        cnt = sc_ref[:, 64:65]
        pooled = sums / jnp.maximum(cnt, 1.0)
        h = jnp.maximum(
            jnp.dot(pooled, w1_ref[...], preferred_element_type=jnp.float32)
            + b1_ref[...], 0.0)
        o_ref[...] = (
            jnp.dot(h, w2_ref[...], preferred_element_type=jnp.float32)
            + b2_ref[...])

    return pl.pallas_call(
        body,
        out_shape=jax.ShapeDtypeStruct((_G, 2), jnp.float32),
    )(sums_cnt, Wl1, bl1.reshape(1, -1), Wl2, bl2.reshape(1, -1))


def kernel(x, edge_index, batch, W1, b1, W2, b2, Wl1, bl1, Wl2, bl2):
    n, f_in = x.shape
    e = edge_index.shape[1]
    n_pad = _NP
    unit = _NC * _NS * _CHUNK
    e_pad = ((e + unit - 1) // unit) * unit
    pad_node = n_pad - 1

    src = jnp.concatenate(
        [edge_index[0], jnp.full((e_pad - e,), pad_node, jnp.int32)])
    dst = jnp.concatenate(
        [edge_index[1], jnp.full((e_pad - e,), pad_node, jnp.int32)])
    x4n = jnp.stack([
        jnp.pad(x[j::4], ((0, _PR - (n - j + 3) // 4), (0, 8 - f_in)))
        for j in range(4)])
    batch_pad = jnp.pad(batch, (0, n_pad - n), constant_values=_G)
    batchrep = jnp.repeat(batch_pad, 32).reshape(_PR, 128)
    W1p = jnp.pad(W1, ((0, 8 - f_in), (0, 0)))

    degp = _sc_degree(dst, n_pad)
    dvp = _tc_dinv(degp.reshape(2 * n_pad // 128, 128))
    dinvrep = jnp.repeat(dvp.reshape(n_pad), 32).reshape(_PR, 128)
    u1_pk = _tc_stage_a(x4n, dinvrep, W1p)
    agg1 = _sc_aggregate(src, dst, u1_pk.reshape(n_pad, 32), 256, False)
    u2_pk = _tc_stage_b(agg1.reshape(2, _PR, 128), u1_pk, dinvrep, b1, W2)
    agg2 = _sc_aggregate(src, dst, u2_pk.reshape(2, n_pad, 32), 256, True)
    sums_cnt = _tc_stage_c(agg2.reshape(2, _PR, 128), u2_pk, dinvrep,
                           batchrep, b2)
    return _tc_head(sums_cnt, Wl1, bl1, Wl2, bl2)


# speedup vs baseline: 1.1349x; 1.1349x over previous
import functools

import jax
import jax.numpy as jnp
from jax import lax
from jax.experimental import pallas as pl
from jax.experimental.pallas import tpu as pltpu
from jax.experimental.pallas import tpu_sc as plsc

_NC = 2
_NS = 16
_CHUNK = 448
_G = 64
_NP = 51200
_PR = _NP // 4


def _mesh():
    return plsc.VectorSubcoreMesh(
        core_axis_name="c", subcore_axis_name="s", num_cores=_NC,
        num_subcores=_NS)


def _fill_zeros_1d(ref, n):
    def body(i, _):
        ref[pl.ds(i * 16, 16)] = jnp.zeros((16,), jnp.float32)
        return 0
    lax.fori_loop(0, n // 16, body, 0)


def _fill_zeros_2d(ref, rows, w):
    def body(i, _):
        for k in range(w // 16):
            ref[i, pl.ds(k * 16, 16)] = jnp.zeros((16,), jnp.float32)
        return 0
    lax.fori_loop(0, rows, body, 0)


def _sc_degree(dst_pad, n_pad):
    e_pad = dst_pad.shape[0]
    per_tile = e_pad // (_NC * _NS)
    n_chunks = per_tile // _CHUNK
    stripe = n_pad // _NS
    zrows = 800
    assert stripe % zrows == 0 and per_tile % _CHUNK == 0

    @functools.partial(
        pl.kernel,
        out_type=jax.ShapeDtypeStruct((_NC, n_pad), jnp.float32),
        mesh=_mesh(),
        scratch_types=[
            pltpu.VMEM((2, _CHUNK), jnp.int32),
            pltpu.VMEM((_CHUNK,), jnp.float32),
            pltpu.VMEM((zrows,), jnp.float32),
            pltpu.VMEM_SHARED((n_pad,), jnp.float32),
            pltpu.SemaphoreType.DMA,
            pltpu.SemaphoreType.DMA,
        ],
        compiler_params=pltpu.CompilerParams(use_tc_tiling_on_sc=False),
    )
    def k(dst_hbm, out_hbm, idx_v, ones_v, zbuf_v, acc_sh, si0, si1):
        c = lax.axis_index("c")
        s = lax.axis_index("s")
        sem_i = [si0, si1]
        for i in range(_CHUNK // 16):
            ones_v[pl.ds(i * 16, 16)] = jnp.ones((16,), jnp.float32)
        _fill_zeros_1d(zbuf_v, zrows)
        row0 = s * stripe

        def zloop(j, _):
            pltpu.sync_copy(zbuf_v, acc_sh.at[pl.ds(row0 + j * zrows, zrows)])
            return 0
        lax.fori_loop(0, stripe // zrows, zloop, 0)
        plsc.subcore_barrier()

        base = (c * _NS + s) * per_tile
        npairs = n_chunks // 2
        assert n_chunks % 2 == 0

        def issue_idx(cc, b):
            pltpu.async_copy(dst_hbm.at[pl.ds(base + cc * _CHUNK, _CHUNK)],
                             idx_v.at[b], sem_i[b])

        def wait_idx(cc, b):
            pltpu.make_async_copy(
                dst_hbm.at[pl.ds(base + cc * _CHUNK, _CHUNK)],
                idx_v.at[b], sem_i[b]).wait()

        issue_idx(0, 0)
        issue_idx(1, 1)

        def body(g, _):
            for half in range(2):
                cc = 2 * g + half
                b = half
                wait_idx(cc, b)
                pltpu.sync_copy(ones_v, acc_sh.at[idx_v.at[b]], add=True)

                @pl.when(g < npairs - 1)
                def _():
                    issue_idx(cc + 2, b)
            return 0
        lax.fori_loop(0, npairs, body, 0)
        plsc.subcore_barrier()
        pltpu.sync_copy(acc_sh.at[pl.ds(row0, stripe)],
                        out_hbm.at[c, pl.ds(row0, stripe)])

    return k(dst_pad)


def _sc_aggregate(src_pad, dst_pad, table, chunk, fsplit):
    e_pad = src_pad.shape[0]
    if fsplit:
        _, n_pad, w = table.shape
        per_tile = e_pad // _NS
    else:
        n_pad, w = table.shape
        per_tile = e_pad // (_NC * _NS)
    n_chunks = per_tile // chunk
    npairs = n_chunks // 2
    stripe = n_pad // _NS
    zrows = 200
    assert stripe % zrows == 0 and per_tile % chunk == 0 and n_chunks % 2 == 0

    @functools.partial(
        pl.kernel,
        out_type=jax.ShapeDtypeStruct((_NC, n_pad, w), jnp.float32),
        mesh=_mesh(),
        scratch_types=[
            pltpu.VMEM((2, chunk), jnp.int32),
            pltpu.VMEM((2, chunk), jnp.int32),
            pltpu.VMEM((2, chunk, w), jnp.float32),
            pltpu.VMEM((zrows, w), jnp.float32),
            pltpu.VMEM_SHARED((n_pad, w), jnp.float32),
            pltpu.SemaphoreType.DMA,
            pltpu.SemaphoreType.DMA,
            pltpu.SemaphoreType.DMA,
            pltpu.SemaphoreType.DMA,
        ],
        compiler_params=pltpu.CompilerParams(use_tc_tiling_on_sc=False),
    )
    def k(src_hbm, dst_hbm, tab_hbm, out_hbm,
          sidx_v, didx_v, rows_v, zbuf_v, acc_sh, sg0, sg1, si0, si1):
        c = lax.axis_index("c")
        s = lax.axis_index("s")
        sem_g = [sg0, sg1]
        sem_i = [si0, si1]
        tab = tab_hbm.at[c] if fsplit else tab_hbm
        _fill_zeros_2d(zbuf_v, zrows, w)
        row0 = s * stripe

        def zloop(j, _):
            pltpu.sync_copy(zbuf_v, acc_sh.at[pl.ds(row0 + j * zrows, zrows)])
            return 0
        lax.fori_loop(0, stripe // zrows, zloop, 0)
        plsc.subcore_barrier()

        base = (s if fsplit else c * _NS + s) * per_tile

        def issue_idx(cc, b):
            off = base + cc * chunk
            pltpu.async_copy(src_hbm.at[pl.ds(off, chunk)], sidx_v.at[b],
                             sem_i[b])
            pltpu.async_copy(dst_hbm.at[pl.ds(off, chunk)], didx_v.at[b],
                             sem_i[b])

        def wait_idx(cc, b):
            off = base + cc * chunk
            pltpu.make_async_copy(src_hbm.at[pl.ds(off, chunk)],
                                  sidx_v.at[b], sem_i[b]).wait()
            pltpu.make_async_copy(dst_hbm.at[pl.ds(off, chunk)],
                                  didx_v.at[b], sem_i[b]).wait()

        def issue_gather(b):
            pltpu.async_copy(tab.at[sidx_v.at[b]], rows_v.at[b], sem_g[b])

        def wait_gather(b):
            pltpu.make_async_copy(tab.at[sidx_v.at[b]], rows_v.at[b],
                                  sem_g[b]).wait()

        issue_idx(0, 0)
        issue_idx(1, 1)
        wait_idx(0, 0)
        issue_gather(0)

        def body(g, _):
            for half in range(2):
                cc = 2 * g + half
                b = half
                nb = 1 - half
                wait_gather(b)
                if half == 0:
                    wait_idx(cc + 1, nb)
                    issue_gather(nb)
                else:
                    @pl.when(g < npairs - 1)
                    def _():
                        wait_idx(cc + 1, nb)
                        issue_gather(nb)
                pltpu.sync_copy(rows_v.at[b], acc_sh.at[didx_v.at[b]],
                                add=True)

                @pl.when(g < npairs - 1)
                def _():
                    issue_idx(cc + 2, b)
            return 0
        lax.fori_loop(0, npairs, body, 0)
        plsc.subcore_barrier()
        pltpu.sync_copy(acc_sh.at[pl.ds(row0, stripe)],
                        out_hbm.at[c, pl.ds(row0, stripe)])

    return k(src_pad, dst_pad, table)


def _tc_dinv(degp_pk):
    m = degp_pk.shape[0] // 2

    def body(d0_ref, d1_ref, o_ref):
        o_ref[...] = lax.rsqrt(d0_ref[...] + d1_ref[...] + 1.0)

    return pl.pallas_call(
        body,
        grid=(1,),
        in_specs=[
            pl.BlockSpec((m, 128), lambda i: (0, 0)),
            pl.BlockSpec((m, 128), lambda i: (1, 0)),
        ],
        out_specs=pl.BlockSpec((m, 128), lambda i: (0, 0)),
        out_shape=jax.ShapeDtypeStruct((m, 128), jnp.float32),
    )(degp_pk, degp_pk)


def _tc_stage_a(x4n, dinvrep, W1p):
    br = 512

    def body(x_ref, dv_ref, w_ref, o_ref):
        xx = jnp.concatenate([x_ref[j] for j in range(4)], axis=0)
        h = jnp.dot(xx, w_ref[...], preferred_element_type=jnp.float32)
        o_ref[...] = jnp.concatenate(
            [h[j * br:(j + 1) * br] for j in range(4)],
            axis=1) * dv_ref[...]

    return pl.pallas_call(
        body,
        grid=(_PR // br,),
        in_specs=[
            pl.BlockSpec((4, br, 8), lambda i: (0, i, 0)),
            pl.BlockSpec((br, 128), lambda i: (i, 0)),
            pl.BlockSpec((8, 32), lambda i: (0, 0)),
        ],
        out_specs=pl.BlockSpec((br, 128), lambda i: (i, 0)),
        out_shape=jax.ShapeDtypeStruct((_PR, 128), jnp.float32),
    )(x4n, dinvrep, W1p)


def _tc_stage_b(agg1_pk, u1_pk, dinvrep, b1, W2):
    br = 512

    def body(a_ref, u_ref, dv_ref, b_ref, w_ref, o_ref):
        dv = dv_ref[...]
        b1p = jnp.concatenate([b_ref[...]] * 4, axis=1)
        h1p = jnp.maximum((a_ref[0] + a_ref[1] + u_ref[...]) * dv + b1p, 0.0)
        w2x = jnp.concatenate([
            jnp.concatenate([
                w_ref[...] if i == j else jnp.zeros((32, 64), jnp.float32)
                for j in range(4)], axis=1)
            for i in range(4)], axis=0)
        h2m = jnp.dot(h1p, w2x, preferred_element_type=jnp.float32)
        p0 = jnp.concatenate(
            [h2m[:, 64 * j:64 * j + 32] for j in range(4)], axis=1) * dv
        p1 = jnp.concatenate(
            [h2m[:, 64 * j + 32:64 * j + 64] for j in range(4)], axis=1) * dv
        o_ref[0] = p0
        o_ref[1] = p1

    return pl.pallas_call(
        body,
        grid=(_PR // br,),
        in_specs=[
            pl.BlockSpec((2, br, 128), lambda i: (0, i, 0)),
            pl.BlockSpec((br, 128), lambda i: (i, 0)),
            pl.BlockSpec((br, 128), lambda i: (i, 0)),
            pl.BlockSpec((1, 32), lambda i: (0, 0)),
            pl.BlockSpec((32, 64), lambda i: (0, 0)),
        ],
        out_specs=pl.BlockSpec((2, br, 128), lambda i: (0, i, 0)),
        out_shape=jax.ShapeDtypeStruct((2, _PR, 128), jnp.float32),
    )(agg1_pk, u1_pk, dinvrep, b1.reshape(1, -1), W2)


def _tc_stage_c(agg2_pk, u2_pk, dinvrep, batchrep, b2):
    def body(a_ref, u_ref, dv_ref, br_ref, b_ref, o_ref):
        dv = dv_ref[...]
        b2p0 = jnp.concatenate([b_ref[:, 0:32]] * 4, axis=1)
        b2p1 = jnp.concatenate([b_ref[:, 32:64]] * 4, axis=1)
        h0 = jnp.maximum((a_ref[0] + u_ref[0]) * dv + b2p0, 0.0)
        h1 = jnp.maximum((a_ref[1] + u_ref[1]) * dv + b2p1, 0.0)
        gids = lax.broadcasted_iota(jnp.int32, (128, _G), 1)
        ones = jnp.ones((128, 8), jnp.float32)
        contrib = jnp.zeros((_G, 72), jnp.float32)
        for j in range(4):
            hj = jnp.concatenate(
                [h0[:, 32 * j:32 * j + 32], h1[:, 32 * j:32 * j + 32], ones],
                axis=1)
            bcol = br_ref[:, 32 * j:32 * j + 1]
            ohj = (bcol == gids).astype(jnp.float32)
            contrib = contrib + lax.dot_general(
                ohj, hj, (((0,), (0,)), ((), ())),
                preferred_element_type=jnp.float32,
                precision=lax.Precision.HIGHEST)

        @pl.when(pl.program_id(0) == 0)
        def _():
            o_ref[...] = jnp.zeros((_G, 72), jnp.float32)
        o_ref[...] += contrib

    return pl.pallas_call(
        body,
        grid=(_PR // 128,),
        in_specs=[
            pl.BlockSpec((2, 128, 128), lambda i: (0, i, 0)),
            pl.BlockSpec((2, 128, 128), lambda i: (0, i, 0)),
            pl.BlockSpec((128, 128), lambda i: (i, 0)),
            pl.BlockSpec((128, 128), lambda i: (i, 0)),
            pl.BlockSpec((1, 64), lambda i: (0, 0)),
        ],
        out_specs=pl.BlockSpec((_G, 72), lambda i: (0, 0)),
        out_shape=jax.ShapeDtypeStruct((_G, 72), jnp.float32),
    )(agg2_pk, u2_pk, dinvrep, batchrep, b2.reshape(1, -1))


def _tc_head(sums_cnt, Wl1, bl1, Wl2, bl2):
    def body(sc_ref, w1_ref, b1_ref, w2_ref, b2_ref, o_ref):
        sums = sc_ref[:, 0:64]
        cnt = sc_ref[:, 64:65]
        pooled = sums / jnp.maximum(cnt, 1.0)
        h = jnp.maximum(
            jnp.dot(pooled, w1_ref[...], preferred_element_type=jnp.float32)
            + b1_ref[...], 0.0)
        o_ref[...] = (
            jnp.dot(h, w2_ref[...], preferred_element_type=jnp.float32)
            + b2_ref[...])

    return pl.pallas_call(
        body,
        out_shape=jax.ShapeDtypeStruct((_G, 2), jnp.float32),
    )(sums_cnt, Wl1, bl1.reshape(1, -1), Wl2, bl2.reshape(1, -1))


def kernel(x, edge_index, batch, W1, b1, W2, b2, Wl1, bl1, Wl2, bl2):
    n, f_in = x.shape
    e = edge_index.shape[1]
    n_pad = _NP
    unit = _NC * _NS * _CHUNK
    e_pad = ((e + unit - 1) // unit) * unit
    pad_node = n_pad - 1

    src = jnp.concatenate(
        [edge_index[0], jnp.full((e_pad - e,), pad_node, jnp.int32)])
    dst = jnp.concatenate(
        [edge_index[1], jnp.full((e_pad - e,), pad_node, jnp.int32)])
    x4n = jnp.stack([
        jnp.pad(x[j::4], ((0, _PR - (n - j + 3) // 4), (0, 8 - f_in)))
        for j in range(4)])
    batch_pad = jnp.pad(batch, (0, n_pad - n), constant_values=_G)
    batchrep = jnp.repeat(batch_pad, 32).reshape(_PR, 128)
    W1p = jnp.pad(W1, ((0, 8 - f_in), (0, 0)))

    degp = _sc_degree(dst, n_pad)
    dvp = _tc_dinv(degp.reshape(2 * n_pad // 128, 128))
    dinvrep = jnp.repeat(dvp.reshape(n_pad), 32).reshape(_PR, 128)
    u1_pk = _tc_stage_a(x4n, dinvrep, W1p)
    agg1 = _sc_aggregate(src, dst, u1_pk.reshape(n_pad, 32), 256, False)
    u2_pk = _tc_stage_b(agg1.reshape(2, _PR, 128), u1_pk, dinvrep, b1, W2)
    agg2 = _sc_aggregate(src, dst, u2_pk.reshape(2, n_pad, 32), 256, True)
    sums_cnt = _tc_stage_c(agg2.reshape(2, _PR, 128), u2_pk, dinvrep,
                           batchrep, b2)
    return _tc_head(sums_cnt, Wl1, bl1, Wl2, bl2)


# speedup vs baseline: 1.2173x; 1.0726x over previous
import functools

import jax
import jax.numpy as jnp
from jax import lax
from jax.experimental import pallas as pl
from jax.experimental.pallas import tpu as pltpu
from jax.experimental.pallas import tpu_sc as plsc

_NC = 2
_NS = 16
_CHUNK = 448
_G = 64
_NP = 51200
_PR = _NP // 4


def _mesh():
    return plsc.VectorSubcoreMesh(
        core_axis_name="c", subcore_axis_name="s", num_cores=_NC,
        num_subcores=_NS)


def _fill_zeros_1d(ref, n):
    def body(i, _):
        ref[pl.ds(i * 16, 16)] = jnp.zeros((16,), jnp.float32)
        return 0
    lax.fori_loop(0, n // 16, body, 0)


def _fill_zeros_2d(ref, rows, w):
    def body(i, _):
        for k in range(w // 16):
            ref[i, pl.ds(k * 16, 16)] = jnp.zeros((16,), jnp.float32)
        return 0
    lax.fori_loop(0, rows, body, 0)


def _sc_degree(dst_pad, n_pad):
    e_pad = dst_pad.shape[0]
    per_tile = e_pad // (_NC * _NS)
    n_chunks = per_tile // _CHUNK
    stripe = n_pad // _NS
    zrows = 800
    assert stripe % zrows == 0 and per_tile % _CHUNK == 0

    @functools.partial(
        pl.kernel,
        out_type=jax.ShapeDtypeStruct((_NC, n_pad), jnp.float32),
        mesh=_mesh(),
        scratch_types=[
            pltpu.VMEM((2, _CHUNK), jnp.int32),
            pltpu.VMEM((_CHUNK,), jnp.float32),
            pltpu.VMEM((zrows,), jnp.float32),
            pltpu.VMEM_SHARED((n_pad,), jnp.float32),
            pltpu.SemaphoreType.DMA,
            pltpu.SemaphoreType.DMA,
        ],
        compiler_params=pltpu.CompilerParams(use_tc_tiling_on_sc=False),
    )
    def k(dst_hbm, out_hbm, idx_v, ones_v, zbuf_v, acc_sh, si0, si1):
        c = lax.axis_index("c")
        s = lax.axis_index("s")
        sem_i = [si0, si1]
        for i in range(_CHUNK // 16):
            ones_v[pl.ds(i * 16, 16)] = jnp.ones((16,), jnp.float32)
        _fill_zeros_1d(zbuf_v, zrows)
        row0 = s * stripe

        def zloop(j, _):
            pltpu.sync_copy(zbuf_v, acc_sh.at[pl.ds(row0 + j * zrows, zrows)])
            return 0
        lax.fori_loop(0, stripe // zrows, zloop, 0)
        plsc.subcore_barrier()

        base = (c * _NS + s) * per_tile
        npairs = n_chunks // 2
        assert n_chunks % 2 == 0

        def issue_idx(cc, b):
            pltpu.async_copy(dst_hbm.at[pl.ds(base + cc * _CHUNK, _CHUNK)],
                             idx_v.at[b], sem_i[b])

        def wait_idx(cc, b):
            pltpu.make_async_copy(
                dst_hbm.at[pl.ds(base + cc * _CHUNK, _CHUNK)],
                idx_v.at[b], sem_i[b]).wait()

        issue_idx(0, 0)
        issue_idx(1, 1)

        def body(g, _):
            for half in range(2):
                cc = 2 * g + half
                b = half
                wait_idx(cc, b)
                pltpu.sync_copy(ones_v, acc_sh.at[idx_v.at[b]], add=True)

                @pl.when(g < npairs - 1)
                def _():
                    issue_idx(cc + 2, b)
            return 0
        lax.fori_loop(0, npairs, body, 0)
        plsc.subcore_barrier()
        pltpu.sync_copy(acc_sh.at[pl.ds(row0, stripe)],
                        out_hbm.at[c, pl.ds(row0, stripe)])

    return k(dst_pad)


def _sc_aggregate(src_pad, dst_pad, table, chunk, fsplit):
    e_pad = src_pad.shape[0]
    if fsplit:
        _, n_pad, w = table.shape
        per_tile = e_pad // _NS
    else:
        n_pad, w = table.shape
        per_tile = e_pad // (_NC * _NS)
    n_chunks = per_tile // chunk
    npairs = n_chunks // 2
    stripe = n_pad // _NS
    zrows = 200
    assert stripe % zrows == 0 and per_tile % chunk == 0 and n_chunks % 2 == 0

    @functools.partial(
        pl.kernel,
        out_type=jax.ShapeDtypeStruct((_NC, n_pad, w), jnp.float32),
        mesh=_mesh(),
        scratch_types=[
            pltpu.VMEM((2, chunk), jnp.int32),
            pltpu.VMEM((2, chunk), jnp.int32),
            pltpu.VMEM((2, chunk, w), jnp.float32),
            pltpu.VMEM((zrows, w), jnp.float32),
            pltpu.VMEM_SHARED((n_pad, w), jnp.float32),
            pltpu.SemaphoreType.DMA,
            pltpu.SemaphoreType.DMA,
            pltpu.SemaphoreType.DMA,
            pltpu.SemaphoreType.DMA,
        ],
        compiler_params=pltpu.CompilerParams(use_tc_tiling_on_sc=False),
    )
    def k(src_hbm, dst_hbm, tab_hbm, out_hbm,
          sidx_v, didx_v, rows_v, zbuf_v, acc_sh, sg0, sg1, si0, si1):
        c = lax.axis_index("c")
        s = lax.axis_index("s")
        sem_g = [sg0, sg1]
        sem_i = [si0, si1]
        tab = tab_hbm.at[c] if fsplit else tab_hbm
        _fill_zeros_2d(zbuf_v, zrows, w)
        row0 = s * stripe

        def zloop(j, _):
            pltpu.sync_copy(zbuf_v, acc_sh.at[pl.ds(row0 + j * zrows, zrows)])
            return 0
        lax.fori_loop(0, stripe // zrows, zloop, 0)
        plsc.subcore_barrier()

        base = (s if fsplit else c * _NS + s) * per_tile

        def issue_idx(cc, b):
            off = base + cc * chunk
            pltpu.async_copy(src_hbm.at[pl.ds(off, chunk)], sidx_v.at[b],
                             sem_i[b])
            pltpu.async_copy(dst_hbm.at[pl.ds(off, chunk)], didx_v.at[b],
                             sem_i[b])

        def wait_idx(cc, b):
            off = base + cc * chunk
            pltpu.make_async_copy(src_hbm.at[pl.ds(off, chunk)],
                                  sidx_v.at[b], sem_i[b]).wait()
            pltpu.make_async_copy(dst_hbm.at[pl.ds(off, chunk)],
                                  didx_v.at[b], sem_i[b]).wait()

        def issue_gather(b):
            pltpu.async_copy(tab.at[sidx_v.at[b]], rows_v.at[b], sem_g[b])

        def wait_gather(b):
            pltpu.make_async_copy(tab.at[sidx_v.at[b]], rows_v.at[b],
                                  sem_g[b]).wait()

        issue_idx(0, 0)
        issue_idx(1, 1)
        wait_idx(0, 0)
        issue_gather(0)

        def body(g, _):
            for half in range(2):
                cc = 2 * g + half
                b = half
                nb = 1 - half
                wait_gather(b)
                if half == 0:
                    wait_idx(cc + 1, nb)
                    issue_gather(nb)
                else:
                    @pl.when(g < npairs - 1)
                    def _():
                        wait_idx(cc + 1, nb)
                        issue_gather(nb)
                pltpu.sync_copy(rows_v.at[b], acc_sh.at[didx_v.at[b]],
                                add=True)

                @pl.when(g < npairs - 1)
                def _():
                    issue_idx(cc + 2, b)
            return 0
        lax.fori_loop(0, npairs, body, 0)
        plsc.subcore_barrier()
        pltpu.sync_copy(acc_sh.at[pl.ds(row0, stripe)],
                        out_hbm.at[c, pl.ds(row0, stripe)])

    return k(src_pad, dst_pad, table)


def _tc_dinv(degp_pk):
    m = degp_pk.shape[0] // 2

    def body(d0_ref, d1_ref, o_ref):
        o_ref[...] = lax.rsqrt(d0_ref[...] + d1_ref[...] + 1.0)

    return pl.pallas_call(
        body,
        grid=(1,),
        in_specs=[
            pl.BlockSpec((m, 128), lambda i: (0, 0)),
            pl.BlockSpec((m, 128), lambda i: (1, 0)),
        ],
        out_specs=pl.BlockSpec((m, 128), lambda i: (0, 0)),
        out_shape=jax.ShapeDtypeStruct((m, 128), jnp.float32),
    )(degp_pk, degp_pk)


def _tc_stage_a(x4n, dinvrep, W1p):
    br = 512

    def body(x_ref, dv_ref, w_ref, o_ref):
        xx = jnp.concatenate([x_ref[j] for j in range(4)], axis=0)
        h = jnp.dot(xx, w_ref[...], preferred_element_type=jnp.float32)
        o_ref[...] = jnp.concatenate(
            [h[j * br:(j + 1) * br] for j in range(4)],
            axis=1) * dv_ref[...]

    return pl.pallas_call(
        body,
        grid=(_PR // br,),
        in_specs=[
            pl.BlockSpec((4, br, 8), lambda i: (0, i, 0)),
            pl.BlockSpec((br, 128), lambda i: (i, 0)),
            pl.BlockSpec((8, 32), lambda i: (0, 0)),
        ],
        out_specs=pl.BlockSpec((br, 128), lambda i: (i, 0)),
        out_shape=jax.ShapeDtypeStruct((_PR, 128), jnp.float32),
    )(x4n, dinvrep, W1p)


def _tc_stage_b(agg1_pk, u1_pk, dinvrep, b1, W2):
    br = 512

    def body(a_ref, u_ref, dv_ref, b_ref, w_ref, o_ref):
        dv = dv_ref[...]
        b1p = jnp.concatenate([b_ref[...]] * 4, axis=1)
        h1p = jnp.maximum((a_ref[0] + a_ref[1] + u_ref[...]) * dv + b1p, 0.0)
        w2x = jnp.concatenate([
            jnp.concatenate([
                w_ref[...] if i == j else jnp.zeros((32, 64), jnp.float32)
                for j in range(4)], axis=1)
            for i in range(4)], axis=0)
        h2m = jnp.dot(h1p, w2x, preferred_element_type=jnp.float32)
        p0 = jnp.concatenate(
            [h2m[:, 64 * j:64 * j + 32] for j in range(4)], axis=1) * dv
        p1 = jnp.concatenate(
            [h2m[:, 64 * j + 32:64 * j + 64] for j in range(4)], axis=1) * dv
        o_ref[0] = p0
        o_ref[1] = p1

    return pl.pallas_call(
        body,
        grid=(_PR // br,),
        in_specs=[
            pl.BlockSpec((2, br, 128), lambda i: (0, i, 0)),
            pl.BlockSpec((br, 128), lambda i: (i, 0)),
            pl.BlockSpec((br, 128), lambda i: (i, 0)),
            pl.BlockSpec((1, 32), lambda i: (0, 0)),
            pl.BlockSpec((32, 64), lambda i: (0, 0)),
        ],
        out_specs=pl.BlockSpec((2, br, 128), lambda i: (0, i, 0)),
        out_shape=jax.ShapeDtypeStruct((2, _PR, 128), jnp.float32),
    )(agg1_pk, u1_pk, dinvrep, b1.reshape(1, -1), W2)


def _tc_stage_c(agg2_pk, u2_pk, dinvrep, batchrep, b2):
    br = 512

    def body(a_ref, u_ref, dv_ref, br_ref, b_ref, o_ref):
        dv = dv_ref[...]
        b2p0 = jnp.concatenate([b_ref[:, 0:32]] * 4, axis=1)
        b2p1 = jnp.concatenate([b_ref[:, 32:64]] * 4, axis=1)
        h0 = jnp.maximum((a_ref[0] + u_ref[0]) * dv + b2p0, 0.0)
        h1 = jnp.maximum((a_ref[1] + u_ref[1]) * dv + b2p1, 0.0)
        gids = lax.broadcasted_iota(jnp.int32, (br, _G), 1)
        ones = jnp.ones((br, 8), jnp.float32)
        contrib = jnp.zeros((_G, 72), jnp.float32)
        for j in range(4):
            hj = jnp.concatenate(
                [h0[:, 32 * j:32 * j + 32], h1[:, 32 * j:32 * j + 32], ones],
                axis=1)
            bcol = br_ref[:, 32 * j:32 * j + 1]
            ohj = (bcol == gids).astype(jnp.float32)
            contrib = contrib + lax.dot_general(
                ohj, hj, (((0,), (0,)), ((), ())),
                preferred_element_type=jnp.float32,
                precision=lax.Precision.HIGHEST)

        @pl.when(pl.program_id(0) == 0)
        def _():
            o_ref[...] = jnp.zeros((_G, 72), jnp.float32)
        o_ref[...] += contrib

    return pl.pallas_call(
        body,
        grid=(_PR // br,),
        in_specs=[
            pl.BlockSpec((2, br, 128), lambda i: (0, i, 0)),
            pl.BlockSpec((2, br, 128), lambda i: (0, i, 0)),
            pl.BlockSpec((br, 128), lambda i: (i, 0)),
            pl.BlockSpec((br, 128), lambda i: (i, 0)),
            pl.BlockSpec((1, 64), lambda i: (0, 0)),
        ],
        out_specs=pl.BlockSpec((_G, 72), lambda i: (0, 0)),
        out_shape=jax.ShapeDtypeStruct((_G, 72), jnp.float32),
    )(agg2_pk, u2_pk, dinvrep, batchrep, b2.reshape(1, -1))


def _tc_head(sums_cnt, Wl1, bl1, Wl2, bl2):
    def body(sc_ref, w1_ref, b1_ref, w2_ref, b2_ref, o_ref):
        sums = sc_ref[:, 0:64]
        cnt = sc_ref[:, 64:65]
        pooled = sums / jnp.maximum(cnt, 1.0)
        h = jnp.maximum(
            jnp.dot(pooled, w1_ref[...], preferred_element_type=jnp.float32)
            + b1_ref[...], 0.0)
        o_ref[...] = (
            jnp.dot(h, w2_ref[...], preferred_element_type=jnp.float32)
            + b2_ref[...])

    return pl.pallas_call(
        body,
        out_shape=jax.ShapeDtypeStruct((_G, 2), jnp.float32),
    )(sums_cnt, Wl1, bl1.reshape(1, -1), Wl2, bl2.reshape(1, -1))


def kernel(x, edge_index, batch, W1, b1, W2, b2, Wl1, bl1, Wl2, bl2):
    n, f_in = x.shape
    e = edge_index.shape[1]
    n_pad = _NP
    unit = _NC * _NS * _CHUNK
    e_pad = ((e + unit - 1) // unit) * unit
    pad_node = n_pad - 1

    src = jnp.concatenate(
        [edge_index[0], jnp.full((e_pad - e,), pad_node, jnp.int32)])
    dst = jnp.concatenate(
        [edge_index[1], jnp.full((e_pad - e,), pad_node, jnp.int32)])
    x4n = jnp.stack([
        jnp.pad(x[j::4], ((0, _PR - (n - j + 3) // 4), (0, 8 - f_in)))
        for j in range(4)])
    batch_pad = jnp.pad(batch, (0, n_pad - n), constant_values=_G)
    batchrep = jnp.repeat(batch_pad, 32).reshape(_PR, 128)
    W1p = jnp.pad(W1, ((0, 8 - f_in), (0, 0)))

    degp = _sc_degree(dst, n_pad)
    dvp = _tc_dinv(degp.reshape(2 * n_pad // 128, 128))
    dinvrep = jnp.repeat(dvp.reshape(n_pad), 32).reshape(_PR, 128)
    u1_pk = _tc_stage_a(x4n, dinvrep, W1p)
    agg1 = _sc_aggregate(src, dst, u1_pk.reshape(n_pad, 32), 256, False)
    u2_pk = _tc_stage_b(agg1.reshape(2, _PR, 128), u1_pk, dinvrep, b1, W2)
    agg2 = _sc_aggregate(src, dst, u2_pk.reshape(2, n_pad, 32), 256, True)
    sums_cnt = _tc_stage_c(agg2.reshape(2, _PR, 128), u2_pk, dinvrep,
                           batchrep, b2)
    return _tc_head(sums_cnt, Wl1, bl1, Wl2, bl2)
